# Initial kernel scaffold; baseline (speedup 1.0000x reference)
import jax
import jax.numpy as jnp
from jax.experimental import pallas as pl


def kernel(batch_boxes, batch_classes, anchors, batch_gt, batch_num_objects):
    raise NotImplementedError("write your pallas kernel here")


# speedup vs baseline: 4.5912x; 4.5912x over previous
import functools

import jax
import jax.numpy as jnp
from jax import lax
from jax.experimental import pallas as pl
from jax.experimental.pallas import tpu as pltpu

_B = 8
_N = 5000
_NP = 5120
_G = 64
_THRESHOLD = 0.5
_ALPHA_CLASS = 0.01
_ALPHA_COORD = 1.0


def _loss_body(a_ref, b_ref, c_ref, gt_ref, valid_ref, cls_out, crd_out, tot_out):
    ax1 = a_ref[0:1, :]
    ay1 = a_ref[1:2, :]
    ax2 = a_ref[2:3, :]
    ay2 = a_ref[3:4, :]
    area_a = (ax2 - ax1) * (ay2 - ay1)

    idx = lax.broadcasted_iota(jnp.int32, (_G, _NP), 1)

    class_acc = jnp.float32(0.0)
    coord_acc = jnp.float32(0.0)
    for i in range(_B):
        g = gt_ref[i]
        cx = g[:, 0:1]
        cy = g[:, 1:2]
        hw = g[:, 2:3] * 0.5
        hh = g[:, 3:4] * 0.5
        gx1 = cx - hw
        gy1 = cy - hh
        gx2 = cx + hw
        gy2 = cy + hh
        area_b = (gx2 - gx1) * (gy2 - gy1)

        iw = jnp.maximum(jnp.minimum(ax2, gx2) - jnp.maximum(ax1, gx1), 0.0)
        ih = jnp.maximum(jnp.minimum(ay2, gy2) - jnp.maximum(ay1, gy1), 0.0)
        inter = iw * ih
        iou = inter / (area_a + area_b - inter)

        colmax = jnp.max(iou, axis=1, keepdims=True)
        midx = jnp.where(iou == colmax, idx, _NP)
        minidx = jnp.min(midx, axis=1, keepdims=True)
        forced = idx == minidx

        vb = valid_ref[i] > 0.5
        mask = ((iou > _THRESHOLD) | forced) & vb
        maskf = mask.astype(jnp.float32)

        bx = b_ref[i]
        d = (jnp.abs(bx[0:1, :] - gx1) + jnp.abs(bx[1:2, :] - gy1)
             + jnp.abs(bx[2:3, :] - gx2) + jnp.abs(bx[3:4, :] - gy2))
        ctot = jnp.sum(maskf * d)
        cnt = jnp.sum(maskf) * 4.0
        coord_acc = coord_acc + ctot / cnt

        pos = jnp.any(mask, axis=0, keepdims=True)
        p = jnp.where(pos, c_ref[i, 1:2, :], c_ref[i, 0:1, :])
        omp = 1.0 - p
        focal = -(omp * omp) * jnp.log(p)
        class_acc = class_acc + jnp.sum(focal)

    cls = class_acc * (_ALPHA_CLASS / _B)
    crd = coord_acc * (_ALPHA_COORD / _B)
    cls_out[0, 0] = cls
    crd_out[0, 0] = crd
    tot_out[0, 0] = cls + crd


@jax.jit
def kernel(batch_boxes, batch_classes, anchors, batch_gt, batch_num_objects):
    pad = _NP - _N
    pad_anchor = jnp.tile(
        jnp.array([[-3.0, -3.0, -2.0, -2.0]], dtype=jnp.float32), (pad, 1))
    anchors_t = jnp.concatenate([anchors, pad_anchor], axis=0).T
    boxes_t = jnp.concatenate(
        [batch_boxes,
         jnp.zeros((_B, pad, 4), dtype=jnp.float32)], axis=1).transpose(0, 2, 1)
    classes_t = jnp.concatenate(
        [batch_classes,
         jnp.ones((_B, pad, 2), dtype=jnp.float32)], axis=1).transpose(0, 2, 1)
    valid = (jnp.arange(_G, dtype=jnp.int32)[None, :]
             < batch_num_objects.astype(jnp.int32)[:, None])
    valid = valid.astype(jnp.float32)[..., None]

    out_shape = [jax.ShapeDtypeStruct((1, 1), jnp.float32)] * 3
    cls, crd, tot = pl.pallas_call(
        _loss_body,
        out_shape=out_shape,
        out_specs=[pl.BlockSpec(memory_space=pltpu.SMEM)] * 3,
    )(anchors_t, boxes_t, classes_t, batch_gt, valid)
    return (tot.reshape(1), cls.reshape(1), crd.reshape(1))
